# baseline (speedup 1.0000x reference)
import jax
import jax.numpy as jnp
from jax import lax
from jax.experimental import pallas as pl
from jax.experimental.pallas import tpu as pltpu
from jax.experimental.pallas import tpu_sc as plsc

_NSUB = 16
_LANE = 16
_CHUNK = 128
_CG = 20


def _ih_matmul(x2, w_pad, b_pad, n_tile):
    tb, k = x2.shape
    hp = w_pad.shape[0]
    grid = (hp // n_tile,)

    def body(x_ref, w_ref, b_ref, o_ref):
        acc = lax.dot_general(
            x_ref[...], w_ref[...],
            (((1,), (1,)), ((), ())),
            preferred_element_type=jnp.float32,
        )
        o_ref[...] = acc + b_ref[...]

    return pl.pallas_call(
        body,
        grid=grid,
        in_specs=[
            pl.BlockSpec((tb, k), lambda j: (0, 0)),
            pl.BlockSpec((n_tile, k), lambda j: (j, 0)),
            pl.BlockSpec((1, n_tile), lambda j: (0, j)),
        ],
        out_specs=pl.BlockSpec((tb, n_tile), lambda j: (0, j)),
        out_shape=jax.ShapeDtypeStruct((tb, hp), jnp.float32),
    )(x2, w_pad, b_pad)


def _make_sc_rnn(hp, n_cores, n_chunks, t_steps):
    mesh = plsc.VectorSubcoreMesh(
        core_axis_name="c", subcore_axis_name="s", num_cores=n_cores)
    n_groups = n_chunks // _CG
    rows_per_sub = hp // _NSUB
    n_work = n_cores * _NSUB
    rows_per_comb = hp // n_work

    def body(ih_ref, cols_ref, rows_ref, vals_ref, zeros_ref,
             ys_ref, h_ref, p_ref, flag_ref,
             acc, colsv0, colsv1, rowsv0, rowsv1, valsv0, valsv1,
             gath0, gath1, p0v, p1v, combv, ihv, flagv, fbuf,
             gsem0, gsem1, ssem):
        colsv = (colsv0, colsv1)
        rowsv = (rowsv0, rowsv1)
        valsv = (valsv0, valsv1)
        gath = (gath0, gath1)
        gsem = (gsem0, gsem1)
        cid = lax.axis_index("c")
        sid = lax.axis_index("s")
        w = cid * _NSUB + sid
        slab = pl.ds(sid * rows_per_sub, rows_per_sub)
        wslab = pl.ds(w * rows_per_comb, rows_per_comb)

        def cross_core_sync(q, token):
            plsc.subcore_barrier()

            @pl.when(sid == 0)
            def publish_and_poll():
                flagv[...] = jnp.full((_LANE,), token, jnp.int32)
                pltpu.sync_copy(flagv, flag_ref.at[q, cid])

                def cond(carry):
                    i, done = carry
                    return jnp.logical_and(jnp.logical_not(done),
                                           i < 10_000_000)

                def poll(carry):
                    i, _ = carry
                    pltpu.sync_copy(flag_ref.at[q, 1 - cid], fbuf)
                    ok = jnp.all(fbuf[...] == token)
                    return (i + 1, ok)

                lax.while_loop(cond, poll, (jnp.int32(0), jnp.bool_(False)))

            plsc.subcore_barrier()

        pltpu.sync_copy(zeros_ref.at[slab], acc.at[slab])
        plsc.subcore_barrier()

        def step(t, _):
            @pl.when(t > 0)
            def phase1():
                def fire(g, b):
                    base = g * _CG
                    pltpu.sync_copy(cols_ref.at[w, pl.ds(base, _CG)], colsv[b])
                    pltpu.sync_copy(rows_ref.at[w, pl.ds(base, _CG)], rowsv[b])
                    pltpu.sync_copy(vals_ref.at[w, pl.ds(base, _CG)], valsv[b])
                    for j in range(_CG):
                        pltpu.async_copy(
                            h_ref.at[colsv[b].at[j]], gath[b].at[j], gsem[b])

                fire(0, 0)

                def outer(i, _):
                    g0 = i * 2
                    for b in range(2):
                        g = g0 + b
                        nxt = g + 1

                        @pl.when(nxt < n_groups)
                        def _():
                            fire(nxt, (b + 1) % 2)

                        for j in range(_CG):
                            pltpu.make_async_copy(
                                h_ref.at[colsv[b].at[j]], gath[b].at[j],
                                gsem[b]).wait()
                        scatters = []
                        for j in range(_CG):
                            def mul16(i2, _):
                                e0 = i2 * _LANE
                                vv = valsv[b][j, pl.ds(e0, _LANE)]
                                for u in range(_LANE):
                                    gath[b][j, e0 + u, :] = (
                                        gath[b][j, e0 + u, :] * vv[u])
                                return 0
                            lax.fori_loop(0, _CHUNK // _LANE, mul16, 0,
                                          unroll=2)
                            scatters.append(pltpu.async_copy(
                                gath[b].at[j], acc.at[rowsv[b].at[j]],
                                ssem, add=True))
                        for c in scatters:
                            c.wait()
                    return 0

                lax.fori_loop(0, n_groups // 2, outer, 0)

            plsc.subcore_barrier()

            pltpu.sync_copy(acc.at[slab], p_ref.at[cid, slab])
            zero_cp = pltpu.async_copy(zeros_ref.at[slab], acc.at[slab],
                                       gsem1)
            cross_core_sync(0, 2 * t + 1)
            zero_cp.wait()

            p0_cp = pltpu.async_copy(p_ref.at[0, wslab], p0v, gsem0)
            p1_cp = pltpu.async_copy(p_ref.at[1, wslab], p1v, gsem1)
            ih_cp = pltpu.async_copy(ih_ref.at[t, wslab], ihv, ssem)
            p0_cp.wait()
            p1_cp.wait()
            ih_cp.wait()

            def comb(i, _):
                a = p0v[i, :] + p1v[i, :] + ihv[i, :]
                e2 = jnp.exp(a * 2.0)
                combv[i, :] = 1.0 - 2.0 / (e2 + 1.0)
                return 0

            lax.fori_loop(0, rows_per_comb, comb, 0, unroll=4)
            h_cp = pltpu.async_copy(combv, h_ref.at[wslab], gsem0)
            ys_cp = pltpu.async_copy(combv, ys_ref.at[t, wslab], gsem1)
            h_cp.wait()
            ys_cp.wait()
            cross_core_sync(1, 2 * t + 2)
            return 0

        lax.fori_loop(0, t_steps, step, 0)

    return pl.kernel(
        body,
        out_type=(
            jax.ShapeDtypeStruct((t_steps, hp, _LANE), jnp.float32),
            jax.ShapeDtypeStruct((hp, _LANE), jnp.float32),
            jax.ShapeDtypeStruct((n_cores, hp, _LANE), jnp.float32),
            jax.ShapeDtypeStruct((2, n_cores, _LANE), jnp.int32),
        ),
        mesh=mesh,
        scratch_types=[
            pltpu.VMEM_SHARED((hp, _LANE), jnp.float32),
            pltpu.VMEM((_CG, _CHUNK), jnp.int32),
            pltpu.VMEM((_CG, _CHUNK), jnp.int32),
            pltpu.VMEM((_CG, _CHUNK), jnp.int32),
            pltpu.VMEM((_CG, _CHUNK), jnp.int32),
            pltpu.VMEM((_CG, _CHUNK), jnp.float32),
            pltpu.VMEM((_CG, _CHUNK), jnp.float32),
            pltpu.VMEM((_CG, _CHUNK, _LANE), jnp.float32),
            pltpu.VMEM((_CG, _CHUNK, _LANE), jnp.float32),
            pltpu.VMEM((hp // 32, _LANE), jnp.float32),
            pltpu.VMEM((hp // 32, _LANE), jnp.float32),
            pltpu.VMEM((hp // 32, _LANE), jnp.float32),
            pltpu.VMEM((hp // 32, _LANE), jnp.float32),
            pltpu.VMEM((_LANE,), jnp.int32),
            pltpu.VMEM((_LANE,), jnp.int32),
            pltpu.SemaphoreType.DMA,
            pltpu.SemaphoreType.DMA,
            pltpu.SemaphoreType.DMA,
        ],
        compiler_params=pltpu.CompilerParams(
            use_tc_tiling_on_sc=False, needs_layout_passes=False),
    )


@jax.jit
def kernel(x, hh_indices, hh_values, hh_bias, W_ih):
    b, t, d_in = x.shape
    h_dim = W_ih.shape[0]
    nnz = hh_values.shape[0]
    assert b == _LANE

    n_tile = 2048
    hp = ((h_dim + n_tile - 1) // n_tile) * n_tile
    w_pad = jnp.pad(W_ih, ((0, hp - h_dim), (0, 0)))
    b_pad = jnp.pad(hh_bias.reshape(1, h_dim), ((0, 0), (0, hp - h_dim)))
    x2 = x.swapaxes(0, 1).reshape(t * b, d_in)
    ih2 = _ih_matmul(x2, w_pad, b_pad, n_tile)
    ihs = ih2.reshape(t, b, hp).transpose(0, 2, 1)

    n_cores = 2
    n_work = n_cores * _NSUB
    stride = _CHUNK * _CG * 2
    epw = ((nnz // n_work + stride - 1) // stride) * stride
    ep = epw * n_work
    rows = hh_indices[0]
    cols = hh_indices[1]
    cols_p = jnp.pad(cols, (0, ep - nnz)).reshape(n_work, epw // _CHUNK, _CHUNK)
    rows_p = jnp.pad(rows, (0, ep - nnz)).reshape(n_work, epw // _CHUNK, _CHUNK)
    vals_p = jnp.pad(hh_values, (0, ep - nnz)).reshape(
        n_work, epw // _CHUNK, _CHUNK)

    zeros = jnp.zeros((hp, _LANE), jnp.float32)
    sc_rnn = _make_sc_rnn(hp, n_cores, epw // _CHUNK, t)
    ys, _, _, _ = sc_rnn(ihs, cols_p, rows_p, vals_p, zeros)
    return ys[:, :h_dim, :].transpose(2, 0, 1)


# speedup vs baseline: 1.0063x; 1.0063x over previous
import jax
import jax.numpy as jnp
from jax import lax
from jax.experimental import pallas as pl
from jax.experimental.pallas import tpu as pltpu
from jax.experimental.pallas import tpu_sc as plsc

_NSUB = 16
_LANE = 16
_CHUNK = 128
_CG = 16


def _ih_matmul(x2, w_pad, b_pad, n_tile):
    tb, k = x2.shape
    hp = w_pad.shape[0]
    grid = (hp // n_tile,)

    def body(x_ref, w_ref, b_ref, o_ref):
        acc = lax.dot_general(
            x_ref[...], w_ref[...],
            (((1,), (1,)), ((), ())),
            preferred_element_type=jnp.float32,
        )
        o_ref[...] = acc + b_ref[...]

    return pl.pallas_call(
        body,
        grid=grid,
        in_specs=[
            pl.BlockSpec((tb, k), lambda j: (0, 0)),
            pl.BlockSpec((n_tile, k), lambda j: (j, 0)),
            pl.BlockSpec((1, n_tile), lambda j: (0, j)),
        ],
        out_specs=pl.BlockSpec((tb, n_tile), lambda j: (0, j)),
        out_shape=jax.ShapeDtypeStruct((tb, hp), jnp.float32),
    )(x2, w_pad, b_pad)


def _make_sc_rnn(hp, n_cores, n_chunks, t_steps):
    mesh = plsc.VectorSubcoreMesh(
        core_axis_name="c", subcore_axis_name="s", num_cores=n_cores)
    n_groups = n_chunks // _CG
    rows_per_sub = hp // _NSUB
    n_work = n_cores * _NSUB
    rows_per_comb = hp // n_work

    def body(ih_ref, cols_ref, rows_ref, vals_ref, zeros_ref,
             ys_ref, h_ref, p_ref, flag_ref,
             acc, colsv0, colsv1, rowsv0, rowsv1, valsv0, valsv1,
             gath0, gath1, p0v, p1v, combv, ihv, flagv, fbuf,
             gsem0, gsem1, ssem):
        colsv = (colsv0, colsv1)
        rowsv = (rowsv0, rowsv1)
        valsv = (valsv0, valsv1)
        gath = (gath0, gath1)
        gsem = (gsem0, gsem1)
        cid = lax.axis_index("c")
        sid = lax.axis_index("s")
        w = cid * _NSUB + sid
        slab = pl.ds(sid * rows_per_sub, rows_per_sub)
        wslab = pl.ds(w * rows_per_comb, rows_per_comb)

        def cross_core_sync(q, token):
            plsc.subcore_barrier()

            @pl.when(sid == 0)
            def publish_and_poll():
                flagv[...] = jnp.full((_LANE,), token, jnp.int32)
                pltpu.sync_copy(flagv, flag_ref.at[q, cid])

                def cond(carry):
                    i, done = carry
                    return jnp.logical_and(jnp.logical_not(done),
                                           i < 10_000_000)

                def poll(carry):
                    i, _ = carry
                    pltpu.sync_copy(flag_ref.at[q, 1 - cid], fbuf)
                    ok = jnp.all(fbuf[...] == token)
                    return (i + 1, ok)

                lax.while_loop(cond, poll, (jnp.int32(0), jnp.bool_(False)))

            plsc.subcore_barrier()

        pltpu.sync_copy(zeros_ref.at[slab], acc.at[slab])
        plsc.subcore_barrier()

        def step(t, _):
            @pl.when(t > 0)
            def phase1():
                def fire(g, b):
                    base = g * _CG
                    pltpu.sync_copy(cols_ref.at[w, pl.ds(base, _CG)], colsv[b])
                    pltpu.sync_copy(rows_ref.at[w, pl.ds(base, _CG)], rowsv[b])
                    pltpu.sync_copy(vals_ref.at[w, pl.ds(base, _CG)], valsv[b])
                    for j in range(_CG):
                        pltpu.async_copy(
                            h_ref.at[colsv[b].at[j]], gath[b].at[j], gsem[b])

                fire(0, 0)

                def outer(i, _):
                    g0 = i * 2
                    for b in range(2):
                        g = g0 + b
                        nxt = g + 1

                        @pl.when(nxt < n_groups)
                        def _():
                            fire(nxt, (b + 1) % 2)

                        for j in range(_CG):
                            pltpu.make_async_copy(
                                h_ref.at[colsv[b].at[j]], gath[b].at[j],
                                gsem[b]).wait()
                        scatters = []
                        for j in range(_CG):
                            def mul16(i2, _):
                                e0 = i2 * _LANE
                                vv = valsv[b][j, pl.ds(e0, _LANE)]
                                for u in range(_LANE):
                                    gath[b][j, e0 + u, :] = (
                                        gath[b][j, e0 + u, :] * vv[u])
                                return 0
                            lax.fori_loop(0, _CHUNK // _LANE, mul16, 0,
                                          unroll=2)
                            scatters.append(pltpu.async_copy(
                                gath[b].at[j], acc.at[rowsv[b].at[j]],
                                ssem, add=True))
                        for c in scatters:
                            c.wait()
                    return 0

                lax.fori_loop(0, n_groups // 2, outer, 0)

            plsc.subcore_barrier()

            pltpu.sync_copy(acc.at[slab], p_ref.at[cid, slab])
            zero_cp = pltpu.async_copy(zeros_ref.at[slab], acc.at[slab],
                                       gsem1)
            cross_core_sync(0, 2 * t + 1)
            zero_cp.wait()

            p0_cp = pltpu.async_copy(p_ref.at[0, wslab], p0v, gsem0)
            p1_cp = pltpu.async_copy(p_ref.at[1, wslab], p1v, gsem1)
            ih_cp = pltpu.async_copy(ih_ref.at[t, wslab], ihv, ssem)
            p0_cp.wait()
            p1_cp.wait()
            ih_cp.wait()

            def comb(i, _):
                a = p0v[i, :] + p1v[i, :] + ihv[i, :]
                e2 = jnp.exp(a * 2.0)
                combv[i, :] = 1.0 - 2.0 / (e2 + 1.0)
                return 0

            lax.fori_loop(0, rows_per_comb, comb, 0, unroll=4)
            h_cp = pltpu.async_copy(combv, h_ref.at[wslab], gsem0)
            ys_cp = pltpu.async_copy(combv, ys_ref.at[t, wslab], gsem1)
            h_cp.wait()
            ys_cp.wait()
            cross_core_sync(1, 2 * t + 2)
            return 0

        lax.fori_loop(0, t_steps, step, 0)

    return pl.kernel(
        body,
        out_type=(
            jax.ShapeDtypeStruct((t_steps, hp, _LANE), jnp.float32),
            jax.ShapeDtypeStruct((hp, _LANE), jnp.float32),
            jax.ShapeDtypeStruct((n_cores, hp, _LANE), jnp.float32),
            jax.ShapeDtypeStruct((2, n_cores, _LANE), jnp.int32),
        ),
        mesh=mesh,
        scratch_types=[
            pltpu.VMEM_SHARED((hp, _LANE), jnp.float32),
            pltpu.VMEM((_CG, _CHUNK), jnp.int32),
            pltpu.VMEM((_CG, _CHUNK), jnp.int32),
            pltpu.VMEM((_CG, _CHUNK), jnp.int32),
            pltpu.VMEM((_CG, _CHUNK), jnp.int32),
            pltpu.VMEM((_CG, _CHUNK), jnp.float32),
            pltpu.VMEM((_CG, _CHUNK), jnp.float32),
            pltpu.VMEM((_CG, _CHUNK, _LANE), jnp.float32),
            pltpu.VMEM((_CG, _CHUNK, _LANE), jnp.float32),
            pltpu.VMEM((hp // 32, _LANE), jnp.float32),
            pltpu.VMEM((hp // 32, _LANE), jnp.float32),
            pltpu.VMEM((hp // 32, _LANE), jnp.float32),
            pltpu.VMEM((hp // 32, _LANE), jnp.float32),
            pltpu.VMEM((_LANE,), jnp.int32),
            pltpu.VMEM((_LANE,), jnp.int32),
            pltpu.SemaphoreType.DMA,
            pltpu.SemaphoreType.DMA,
            pltpu.SemaphoreType.DMA,
        ],
        compiler_params=pltpu.CompilerParams(
            use_tc_tiling_on_sc=False, needs_layout_passes=False),
    )


@jax.jit
def kernel(x, hh_indices, hh_values, hh_bias, W_ih):
    b, t, d_in = x.shape
    h_dim = W_ih.shape[0]
    nnz = hh_values.shape[0]
    assert b == _LANE

    n_tile = 2048
    hp = ((h_dim + n_tile - 1) // n_tile) * n_tile
    w_pad = jnp.pad(W_ih, ((0, hp - h_dim), (0, 0)))
    b_pad = jnp.pad(hh_bias.reshape(1, h_dim), ((0, 0), (0, hp - h_dim)))
    x2 = x.swapaxes(0, 1).reshape(t * b, d_in)
    ih2 = _ih_matmul(x2, w_pad, b_pad, n_tile)
    ihs = ih2.reshape(t, b, hp).transpose(0, 2, 1)

    n_cores = 2
    n_work = n_cores * _NSUB
    stride = _CHUNK * _CG * 2
    epw = ((nnz // n_work + stride - 1) // stride) * stride
    ep = epw * n_work
    rows = hh_indices[0]
    cols = hh_indices[1]
    cols_p = jnp.pad(cols, (0, ep - nnz)).reshape(n_work, epw // _CHUNK, _CHUNK)
    rows_p = jnp.pad(rows, (0, ep - nnz)).reshape(n_work, epw // _CHUNK, _CHUNK)
    vals_p = jnp.pad(hh_values, (0, ep - nnz)).reshape(
        n_work, epw // _CHUNK, _CHUNK)

    zeros = jnp.zeros((hp, _LANE), jnp.float32)
    sc_rnn = _make_sc_rnn(hp, n_cores, epw // _CHUNK, t)
    ys, _, _, _ = sc_rnn(ihs, cols_p, rows_p, vals_p, zeros)
    return ys[:, :h_dim, :].transpose(2, 0, 1)
